# baseline (speedup 1.0000x reference)
import functools

import numpy as np
import jax
import jax.numpy as jnp
from jax import lax
from jax.experimental import pallas as pl
from jax.experimental.pallas import tpu as pltpu
from jax.experimental.pallas import tpu_sc as plsc

_NC = 2
_NS = 16


def _posenc_np(seq_len, d_model):
    position = np.arange(seq_len, dtype=np.float32)[:, None]
    div_term = np.exp(
        np.arange(0, d_model, 2, dtype=np.float32) * (-np.log(10000.0) / d_model)
    )
    pe = np.zeros((seq_len, d_model), dtype=np.float32)
    pe[:, 0::2] = np.sin(position * div_term)
    pe[:, 1::2] = np.cos(position * div_term)
    return pe


@functools.lru_cache(maxsize=None)
def _make_sc_kernel(B, N, M, D):
    NBG, NMG = 8, 4
    assert B % NBG == 0 and M % NMG == 0 and D % 16 == 0
    BPW = B // NBG
    MS = M // NMG
    MSP = ((MS + 127) // 128) * 128
    NG = MSP // 16
    H = MSP // 128
    JD = D // 16

    mesh = plsc.VectorSubcoreMesh(core_axis_name="c", subcore_axis_name="s")

    @functools.partial(
        pl.kernel,
        out_type=jax.ShapeDtypeStruct((B * M, D), jnp.float32),
        mesh=mesh,
        compiler_params=pltpu.CompilerParams(use_tc_tiling_on_sc=False,
                                              needs_layout_passes=False),
        scratch_types=[
            pltpu.VMEM((MSP, D), jnp.float32),
            pltpu.VMEM((BPW, D), jnp.float32),
            pltpu.VMEM((D,), jnp.float32),
            pltpu.VMEM((6, D), jnp.float32),
            pltpu.VMEM((MSP, 14), jnp.float32),
            pltpu.VMEM((2 * H, 128), jnp.int32),
            pltpu.VMEM((MSP, D), jnp.float32),
            pltpu.VMEM((MSP, D), jnp.float32),
            pltpu.SemaphoreType.DMA,
            pltpu.SemaphoreType.DMA,
        ],
    )
    def kern(cities, state, graph, bps, w6, pe, out,
             pe_v, graph_v, bps_v, w_v, state_v, idx_v, rows0_v, rows1_v,
             sem0, sem1):
        cid = lax.axis_index("c")
        sid = lax.axis_index("s")
        wid = sid * _NC + cid
        bg = wid // NMG
        mg = wid % NMG
        b_lo = bg * BPW
        m_lo = mg * MS

        pltpu.sync_copy(pe.at[pl.ds(m_lo, MS)], pe_v.at[pl.ds(0, MS)])
        pltpu.sync_copy(graph.at[pl.ds(b_lo, BPW)], graph_v)
        pltpu.sync_copy(bps, bps_v)
        pltpu.sync_copy(w6, w_v)

        lane = lax.iota(jnp.int32, 16)
        wvals = [[w_v[k, pl.ds(j * 16, 16)] for j in range(JD)] for k in range(6)]
        bvals = [bps_v[pl.ds(j * 16, 16)] for j in range(JD)]

        def b_body(bi, carry):
            b = b_lo + bi
            abase = b * M + m_lo

            pltpu.sync_copy(state.at[pl.ds(abase, MS)], state_v.at[pl.ds(0, MS)])

            for g in range(NG):
                rowv = jnp.minimum(g * 16 + lane, MS - 1)
                f0 = plsc.load_gather(state_v, [rowv, jnp.full((16,), 0, jnp.int32)])
                f1 = plsc.load_gather(state_v, [rowv, jnp.full((16,), 1, jnp.int32)])
                i0 = jnp.clip(f0.astype(jnp.int32), 0, N - 1) + b * N
                i1 = jnp.clip(f1.astype(jnp.int32), 0, N - 1) + b * N
                h, c0 = divmod(g * 16, 128)
                idx_v[h, pl.ds(c0, 16)] = i0
                idx_v[H + h, pl.ds(c0, 16)] = i1

            copies = []
            for h in range(H):
                copies.append(pltpu.async_copy(
                    cities.at[idx_v.at[h]], rows0_v.at[pl.ds(h * 128, 128)], sem0))
                copies.append(pltpu.async_copy(
                    cities.at[idx_v.at[H + h]], rows1_v.at[pl.ds(h * 128, 128)], sem1))
            for cp in copies:
                cp.wait()

            base = [graph_v[bi, pl.ds(j * 16, 16)] + bvals[j] for j in range(JD)]

            def g_body(g, inner):
                gbase = g * 16
                cv = [plsc.load_gather(
                          state_v, [gbase + lane, jnp.full((16,), kk + 2, jnp.int32)])
                      for kk in range(6)]
                for ap in range(16):
                    a = gbase + ap
                    cs = [cv[kk][ap] for kk in range(6)]
                    for j in range(JD):
                        sl = pl.ds(j * 16, 16)
                        acc = rows0_v[a, sl] + rows1_v[a, sl]
                        acc = acc + base[j] + pe_v[a, sl]
                        acc = (acc
                               + cs[0] * wvals[0][j] + cs[1] * wvals[1][j]
                               + cs[2] * wvals[2][j] + cs[3] * wvals[3][j]
                               + cs[4] * wvals[4][j] + cs[5] * wvals[5][j])
                        rows0_v[a, sl] = acc
                return inner

            lax.fori_loop(0, NG, g_body, 0)

            pltpu.sync_copy(rows0_v.at[pl.ds(0, MS)], out.at[pl.ds(abase, MS)])
            return carry

        lax.fori_loop(0, BPW, b_body, 0)

    return kern


def kernel(cities_embed, graph_embed, agent_state, W_dc, W_nc, W_ps, b_ps):
    B, N, D = cities_embed.shape
    M = agent_state.shape[1]
    cities = cities_embed.reshape(B * N, D)
    state = agent_state.reshape(B * M, 14)
    graph = graph_embed.reshape(B, D)
    w6 = jnp.concatenate([W_dc, W_nc, W_ps], axis=1).T
    pe = jnp.asarray(_posenc_np(M, D))
    out = _make_sc_kernel(B, N, M, D)(cities, state, graph, b_ps, w6, pe)
    return out.reshape(B, M, D)


# speedup vs baseline: 1.0341x; 1.0341x over previous
import functools

import numpy as np
import jax
import jax.numpy as jnp
from jax import lax
from jax.experimental import pallas as pl
from jax.experimental.pallas import tpu as pltpu
from jax.experimental.pallas import tpu_sc as plsc

_NC = 2
_NS = 16


def _posenc_np(seq_len, d_model):
    position = np.arange(seq_len, dtype=np.float32)[:, None]
    div_term = np.exp(
        np.arange(0, d_model, 2, dtype=np.float32) * (-np.log(10000.0) / d_model)
    )
    pe = np.zeros((seq_len, d_model), dtype=np.float32)
    pe[:, 0::2] = np.sin(position * div_term)
    pe[:, 1::2] = np.cos(position * div_term)
    return pe


@functools.lru_cache(maxsize=None)
def _make_sc_gather(B, N, M, D):
    NBG, NMG = 8, 4
    assert B % NBG == 0 and M % NMG == 0 and D % 16 == 0
    BPW = B // NBG
    MS = M // NMG
    MSP = ((MS + 127) // 128) * 128
    NG = MSP // 16
    H = MSP // 128
    JD = D // 16

    mesh = plsc.VectorSubcoreMesh(core_axis_name="c", subcore_axis_name="s")

    @functools.partial(
        pl.kernel,
        out_type=jax.ShapeDtypeStruct((B * M, D), jnp.float32),
        mesh=mesh,
        compiler_params=pltpu.CompilerParams(use_tc_tiling_on_sc=False,
                                             needs_layout_passes=False),
        scratch_types=[
            pltpu.VMEM((MSP, 14), jnp.float32),
            pltpu.VMEM((2 * H, 128), jnp.int32),
            pltpu.VMEM((MSP, D), jnp.float32),
            pltpu.VMEM((MSP, D), jnp.float32),
            pltpu.SemaphoreType.DMA,
            pltpu.SemaphoreType.DMA,
        ],
    )
    def kern(cities, state, out, state_v, idx_v, rows0_v, rows1_v, sem0, sem1):
        cid = lax.axis_index("c")
        sid = lax.axis_index("s")
        wid = sid * _NC + cid
        bg = wid // NMG
        mg = wid % NMG
        b_lo = bg * BPW
        m_lo = mg * MS

        lane = lax.iota(jnp.int32, 16)

        def b_body(bi, carry):
            b = b_lo + bi
            abase = b * M + m_lo

            pltpu.sync_copy(state.at[pl.ds(abase, MS)], state_v.at[pl.ds(0, MS)])

            for g in range(NG):
                rowv = jnp.minimum(g * 16 + lane, MS - 1)
                f0 = plsc.load_gather(state_v, [rowv, jnp.full((16,), 0, jnp.int32)])
                f1 = plsc.load_gather(state_v, [rowv, jnp.full((16,), 1, jnp.int32)])
                i0 = jnp.clip(f0.astype(jnp.int32), 0, N - 1) + b * N
                i1 = jnp.clip(f1.astype(jnp.int32), 0, N - 1) + b * N
                h, c0 = divmod(g * 16, 128)
                idx_v[h, pl.ds(c0, 16)] = i0
                idx_v[H + h, pl.ds(c0, 16)] = i1

            copies = []
            for h in range(H):
                copies.append(pltpu.async_copy(
                    cities.at[idx_v.at[h]], rows0_v.at[pl.ds(h * 128, 128)], sem0))
                copies.append(pltpu.async_copy(
                    cities.at[idx_v.at[H + h]], rows1_v.at[pl.ds(h * 128, 128)], sem1))
            for cp in copies:
                cp.wait()

            @plsc.parallel_loop(0, MS, unroll=4)
            def comb(a):
                for j in range(JD):
                    sl = pl.ds(j * 16, 16)
                    rows0_v[a, sl] = rows0_v[a, sl] + rows1_v[a, sl]

            pltpu.sync_copy(rows0_v.at[pl.ds(0, MS)], out.at[pl.ds(abase, MS)])
            return carry

        lax.fori_loop(0, BPW, b_body, 0)

    return kern


@functools.lru_cache(maxsize=None)
def _make_tc_combine(B, M, D):
    grid = (B,)

    def body(g_ref, s_ref, gr_ref, bps_ref, w_ref, pe_ref, o_ref):
        lin = lax.dot_general(s_ref[0], w_ref[...],
                              (((0,), (0,)), ((), ())),
                              preferred_element_type=jnp.float32)
        o_ref[0] = g_ref[0] + lin + pe_ref[...] + gr_ref[0] + bps_ref[0]

    return pl.pallas_call(
        body,
        grid=grid,
        in_specs=[
            pl.BlockSpec((1, M, D), lambda b: (b, 0, 0)),
            pl.BlockSpec((1, 8, M), lambda b: (b, 0, 0)),
            pl.BlockSpec((1, 1, D), lambda b: (b, 0, 0)),
            pl.BlockSpec((1, D), lambda b: (0, 0)),
            pl.BlockSpec((8, D), lambda b: (0, 0)),
            pl.BlockSpec((M, D), lambda b: (0, 0)),
        ],
        out_specs=pl.BlockSpec((1, M, D), lambda b: (b, 0, 0)),
        out_shape=jax.ShapeDtypeStruct((B, M, D), jnp.float32),
    )


def kernel(cities_embed, graph_embed, agent_state, W_dc, W_nc, W_ps, b_ps):
    B, N, D = cities_embed.shape
    M = agent_state.shape[1]
    cities = cities_embed.reshape(B * N, D)
    state = agent_state.reshape(B * M, 14)

    gsum = _make_sc_gather(B, N, M, D)(cities, state)

    s8t = agent_state[:, :, 2:10].transpose(0, 2, 1)
    w8 = jnp.concatenate(
        [W_dc, W_nc, W_ps, jnp.zeros((D, 2), jnp.float32)], axis=1).T
    pe = jnp.asarray(_posenc_np(M, D))
    out = _make_tc_combine(B, M, D)(
        gsum.reshape(B, M, D), s8t, graph_embed,
        b_ps.reshape(1, D), w8, pe)
    return out
